# Initial kernel scaffold; baseline (speedup 1.0000x reference)
import jax
import jax.numpy as jnp
from jax.experimental import pallas as pl


def kernel(x, edge_index, edge_attr, params):
    raise NotImplementedError("write your pallas kernel here")


# speedup vs baseline: 2.1192x; 2.1192x over previous
import functools

import jax
import jax.numpy as jnp
from jax import lax
from jax.experimental import pallas as pl
from jax.experimental.pallas import tpu as pltpu
from jax.experimental.pallas import tpu_sc as plsc

N_NODES = 10000
N_EDGES = 320000
D_IN = 128
D_EDGE = 16
HIDDEN = 128
HALF = 64
EPS = 1e-7
LN_EPS = 1e-5

NS = 16
E_PER_S = N_EDGES // NS
CHUNK = 80
N_CHUNK = E_PER_S // CHUNK
STRIPE = 640
FIN_CHUNK = 40

f32 = jnp.float32


def _ln_relu(v, w, b):
    m = jnp.mean(v)
    s = jnp.sqrt(jnp.mean((v - m) ** 2))
    return jnp.maximum((v - m) / (s + LN_EPS) * w + b, 0.0)


def _encode_body(x_ref, w_ref, b_ref, nw_ref, nb_ref, h_ref, hn_ref):
    h = jnp.dot(x_ref[...], w_ref[...], preferred_element_type=f32) + b_ref[...]
    h_ref[...] = h
    hn_ref[...] = _ln_relu(h, nw_ref[0, 0], nb_ref[0, 0])


def _encode(x, w, b, nw, nb):
    return pl.pallas_call(
        _encode_body,
        out_shape=[
            jax.ShapeDtypeStruct((N_NODES, HIDDEN), f32),
            jax.ShapeDtypeStruct((N_NODES, HIDDEN), f32),
        ],
    )(x, w, b.reshape(1, -1), nw.reshape(1, 1), nb.reshape(1, 1))


EDGE_BLK = 8000


def _edge_enc_body(ea_ref, w_ref, b_ref, ew_ref):
    ew = jnp.dot(ea_ref[...], w_ref[...], preferred_element_type=f32) + b_ref[...]
    ew_ref[0] = ew[:, :HALF]
    ew_ref[1] = ew[:, HALF:]


def _edge_encode(ea, w, b):
    nblk = N_EDGES // EDGE_BLK
    return pl.pallas_call(
        _edge_enc_body,
        grid=(nblk,),
        in_specs=[
            pl.BlockSpec((EDGE_BLK, D_EDGE), lambda i: (i, 0)),
            pl.BlockSpec((D_EDGE, HIDDEN), lambda i: (0, 0)),
            pl.BlockSpec((1, HIDDEN), lambda i: (0, 0)),
        ],
        out_specs=pl.BlockSpec((2, EDGE_BLK, HALF), lambda i: (0, i, 0)),
        out_shape=jax.ShapeDtypeStruct((2, N_EDGES, HALF), f32),
    )(ea, w, b.reshape(1, -1))


def _mlp_body(h_ref, o_ref, w1a_ref, w1b_ref, b1_ref, lnw_ref, lnb_ref,
              w2_ref, b2_ref, nnw_ref, nnb_ref, hnew_ref, hn_ref):
    h1 = (jnp.dot(o_ref[0], w1a_ref[...], preferred_element_type=f32)
          + jnp.dot(o_ref[1], w1b_ref[...], preferred_element_type=f32)
          + b1_ref[...])
    h1 = _ln_relu(h1, lnw_ref[0, 0], lnb_ref[0, 0])
    hnew = h_ref[...] + jnp.dot(h1, w2_ref[...], preferred_element_type=f32) + b2_ref[...]
    hnew_ref[...] = hnew
    hn_ref[...] = _ln_relu(hnew, nnw_ref[0, 0], nnb_ref[0, 0])


def _mlp(h, o, lp, next_nw, next_nb):
    return pl.pallas_call(
        _mlp_body,
        out_shape=[
            jax.ShapeDtypeStruct((N_NODES, HIDDEN), f32),
            jax.ShapeDtypeStruct((N_NODES, HIDDEN), f32),
        ],
    )(h, o, lp['mlp_W1'][:HALF], lp['mlp_W1'][HALF:],
      lp['mlp_b1'].reshape(1, -1),
      lp['mlp_ln_w'].reshape(1, 1), lp['mlp_ln_b'].reshape(1, 1),
      lp['mlp_W2'], lp['mlp_b2'].reshape(1, -1),
      next_nw.reshape(1, 1), next_nb.reshape(1, 1))


def _final_body(hn_ref, w_ref, b_ref, y_ref):
    y_ref[...] = (jnp.dot(hn_ref[...], w_ref[...], preferred_element_type=f32)
                  + b_ref[...])


def _final(hn, w, b):
    return pl.pallas_call(
        _final_body,
        out_shape=jax.ShapeDtypeStruct((N_NODES, HIDDEN), f32),
    )(hn, w, b.reshape(1, -1))


_SC_MESH = plsc.VectorSubcoreMesh(core_axis_name="c", subcore_axis_name="s")


@functools.partial(
    pl.kernel,
    out_type=jax.ShapeDtypeStruct((2 * N_NODES, HALF), f32),
    mesh=_SC_MESH,
    scratch_types=[
        pltpu.VMEM_SHARED((N_NODES, 2 * HALF), f32),
        pltpu.VMEM((CHUNK,), jnp.int32),
        pltpu.VMEM((CHUNK,), jnp.int32),
        pltpu.VMEM((CHUNK, HIDDEN), f32),
        pltpu.VMEM((CHUNK, HALF), f32),
        pltpu.VMEM((CHUNK, 2 * HALF), f32),
        pltpu.VMEM((FIN_CHUNK, 2 * HALF), f32),
        pltpu.VMEM((FIN_CHUNK, HIDDEN), f32),
        pltpu.VMEM((FIN_CHUNK, HALF), f32),
        pltpu.VMEM((16,), f32),
        pltpu.SemaphoreType.DMA,
    ],
)
def _sc_msgpass(hn_hbm, ew_hbm, src_hbm, dst_hbm, t_hbm, out_hbm,
                acc_sh, sbuf, dbuf, gbuf, ebuf, cbuf,
                fabuf, fhbuf, fobuf, tbuf, sem):
    c = lax.axis_index("c")
    s = lax.axis_index("s")
    row0 = s * STRIPE
    crow = c * N_NODES
    n_fin = jnp.where(s == NS - 1, 10, STRIPE // FIN_CHUNK)

    def _zero_row(r, _):
        for f in range(8):
            fabuf[r, pl.ds(16 * f, 16)] = jnp.zeros((16,), f32)
        return 0
    lax.fori_loop(0, FIN_CHUNK, _zero_row, 0)

    def _stage(k, _):
        r0 = pl.multiple_of(row0 + k * FIN_CHUNK, 8)
        pltpu.sync_copy(fabuf, acc_sh.at[pl.ds(r0, FIN_CHUNK), :])
        return 0
    lax.fori_loop(0, n_fin, _stage, 0)

    pltpu.sync_copy(t_hbm, tbuf)
    plsc.subcore_barrier()
    tv = tbuf[...]

    def chunk_body(i, _):
        e0 = pl.multiple_of(s * E_PER_S + i * CHUNK, 8)
        pltpu.sync_copy(src_hbm.at[pl.ds(e0, CHUNK)], sbuf)
        pltpu.sync_copy(dst_hbm.at[pl.ds(e0, CHUNK)], dbuf)
        pltpu.async_copy(hn_hbm.at[sbuf], gbuf, sem).wait()
        pltpu.sync_copy(ew_hbm.at[c, pl.ds(e0, CHUNK), :], ebuf)

        def edge_body(j, _):
            for f in range(4):
                hv = gbuf[j, pl.ds(c * HALF + 16 * f, 16)]
                ev = ebuf[j, pl.ds(16 * f, 16)]
                msg = jnp.maximum(hv + ev, 0.0) + EPS
                ex = jnp.exp(msg * tv)
                cbuf[j, pl.ds(16 * f, 16)] = ex * msg
                cbuf[j, pl.ds(HALF + 16 * f, 16)] = ex
            return 0
        lax.fori_loop(0, CHUNK, edge_body, 0)

        pltpu.sync_copy(cbuf, acc_sh.at[dbuf], add=True)
        return 0
    lax.fori_loop(0, N_CHUNK, chunk_body, 0)

    plsc.subcore_barrier()

    def _finalize(k, _):
        r0 = pl.multiple_of(row0 + k * FIN_CHUNK, 8)
        fr0 = pl.multiple_of(crow + row0 + k * FIN_CHUNK, 8)
        pltpu.sync_copy(acc_sh.at[pl.ds(r0, FIN_CHUNK), :], fabuf)
        pltpu.sync_copy(hn_hbm.at[pl.ds(r0, FIN_CHUNK), :], fhbuf)

        def fin_body(r, _):
            for f in range(4):
                num = fabuf[r, pl.ds(16 * f, 16)]
                den = fabuf[r, pl.ds(HALF + 16 * f, 16)]
                hv = fhbuf[r, pl.ds(c * HALF + 16 * f, 16)]
                fobuf[r, pl.ds(16 * f, 16)] = hv + num / (den + EPS)
            return 0
        lax.fori_loop(0, FIN_CHUNK, fin_body, 0)
        pltpu.sync_copy(fobuf, out_hbm.at[pl.ds(fr0, FIN_CHUNK), :])
        return 0
    lax.fori_loop(0, n_fin, _finalize, 0)


def kernel(x, edge_index, edge_attr, params):
    src = edge_index[0].astype(jnp.int32)
    dst = edge_index[1].astype(jnp.int32)
    layers = params['layers']

    ew = _edge_encode(edge_attr, params['edge_enc_W'], params['edge_enc_b'])
    h, hn = _encode(x, params['node_enc_W'], params['node_enc_b'],
                    layers[1]['norm_w'], layers[1]['norm_b'])

    for li in (1, 2, 3):
        lp = layers[li]
        nxt = layers[li + 1] if li < 3 else layers[0]
        t16 = jnp.broadcast_to(lp['t'], (16,)).astype(f32)
        o = _sc_msgpass(hn, ew, src, dst, t16)
        h, hn = _mlp(h, o.reshape(2, N_NODES, HALF), lp,
                     nxt['norm_w'], nxt['norm_b'])

    return _final(hn, params['lin_W'], params['lin_b'])
